# baseline (speedup 1.0000x reference)
import functools

import jax
import jax.numpy as jnp
from jax import lax
from jax.experimental import pallas as pl
from jax.experimental.pallas import tpu as pltpu
from jax.experimental.pallas import tpu_sc as plsc


def _att_body(a_ref, e_ref, w1_ref, b1_ref, w2_ref, b2_ref, wh_ref,
              o_ref, i_ref, j_ref):
    h = jnp.dot(a_ref[...], w1_ref[...], preferred_element_type=jnp.float32)
    h = h + b1_ref[...]
    h = jnp.where(h >= 0.0, h, 0.01 * h)
    v = jnp.dot(w2_ref[...], wh_ref[...], preferred_element_type=jnp.float32)
    c = jnp.sum(b2_ref[...] * wh_ref[...][:, 0])
    ht = h.T
    att = jnp.dot(v.T, ht, preferred_element_type=jnp.float32) + c
    o_ref[...] = att.reshape(att.shape[1])
    i_ref[...] = e_ref[0, :]
    j_ref[...] = e_ref[1, :]


def _compute_att(a_ij, edge_index, W1, b1, W2, b2, Wh, block_e, first_block,
                 chunk_e):
    nb = pl.cdiv(chunk_e, block_e)
    return pl.pallas_call(
        _att_body,
        grid=(nb,),
        in_specs=[
            pl.BlockSpec((block_e, a_ij.shape[1]),
                         lambda g: (g + first_block, 0)),
            pl.BlockSpec((2, block_e), lambda g: (0, g + first_block)),
            pl.BlockSpec(W1.shape, lambda g: (0, 0)),
            pl.BlockSpec(b1.shape, lambda g: (0,)),
            pl.BlockSpec(W2.shape, lambda g: (0, 0)),
            pl.BlockSpec(b2.shape, lambda g: (0,)),
            pl.BlockSpec(Wh.shape, lambda g: (0, 0)),
        ],
        out_specs=[
            pl.BlockSpec((block_e,), lambda g: (g,)),
            pl.BlockSpec((block_e,), lambda g: (g,)),
            pl.BlockSpec((block_e,), lambda g: (g,)),
        ],
        out_shape=[
            jax.ShapeDtypeStruct((chunk_e,), jnp.float32),
            jax.ShapeDtypeStruct((chunk_e,), jnp.int32),
            jax.ShapeDtypeStruct((chunk_e,), jnp.int32),
        ],
    )(a_ij, edge_index, W1, b1, W2, b2, Wh)


_LANES = 16
_MAGIC = 0x5F3759DF


def _rsqrt16(x):
    i = plsc.bitcast(x, jnp.int32)
    i = _MAGIC - lax.shift_right_logical(i, 1)
    y = plsc.bitcast(i, jnp.float32)
    hx = 0.5 * x
    y = y * (1.5 - hx * y * y)
    y = y * (1.5 - hx * y * y)
    return y


def _make_sc_edge(n, e, n_workers):
    ew = e // n_workers
    cw = 3 * n
    mesh = plsc.VectorSubcoreMesh(core_axis_name="c", subcore_axis_name="s")

    @functools.partial(
        pl.kernel,
        mesh=mesh,
        compiler_params=pltpu.CompilerParams(needs_layout_passes=False),
        out_type=jax.ShapeDtypeStruct((n_workers, cw), jnp.float32),
        scratch_types=[
            pltpu.VMEM((cw,), jnp.float32),
            pltpu.VMEM((cw,), jnp.float32),
            pltpu.VMEM((ew,), jnp.int32),
            pltpu.VMEM((ew,), jnp.int32),
            pltpu.VMEM((ew,), jnp.float32),
            pltpu.SemaphoreType.DMA,
            pltpu.SemaphoreType.DMA,
            pltpu.SemaphoreType.DMA,
            pltpu.SemaphoreType.DMA,
        ],
    )
    def sc_edge(coords_hbm, i_hbm, j_hbm, att_hbm, out_hbm,
                coords_v, acc_v, i_v, j_v, att_v, s0, s1, s2, s3):
        cid = lax.axis_index("c")
        sid = lax.axis_index("s")
        wid = sid * 2 + cid
        base = pl.multiple_of(wid * ew, 8)

        c0 = pltpu.async_copy(coords_hbm, coords_v, s0)
        c1 = pltpu.async_copy(i_hbm.at[pl.ds(base, ew)], i_v, s1)
        c2 = pltpu.async_copy(j_hbm.at[pl.ds(base, ew)], j_v, s2)
        c3 = pltpu.async_copy(att_hbm.at[pl.ds(base, ew)], att_v, s3)

        zeros = jnp.zeros((_LANES,), jnp.float32)

        @plsc.parallel_loop(0, cw, _LANES, unroll=8)
        def _(off):
            acc_v[pl.ds(off, _LANES)] = zeros

        c0.wait()
        c1.wait()
        c2.wait()
        c3.wait()

        @plsc.parallel_loop(0, ew, _LANES, unroll=8)
        def _(off):
            iv = i_v[pl.ds(off, _LANES)]
            jv = j_v[pl.ds(off, _LANES)]
            av = att_v[pl.ds(off, _LANES)]
            bi = iv * 3
            bj = jv * 3
            xi = plsc.load_gather(coords_v, [bi])
            yi = plsc.load_gather(coords_v, [bi + 1])
            zi = plsc.load_gather(coords_v, [bi + 2])
            xj = plsc.load_gather(coords_v, [bj])
            yj = plsc.load_gather(coords_v, [bj + 1])
            zj = plsc.load_gather(coords_v, [bj + 2])
            dx = xi - xj
            dy = yi - yj
            dz = zi - zj
            s2 = dx * dx + dy * dy + dz * dz
            s2 = jnp.maximum(s2, 1e-24)
            f = av * _rsqrt16(s2)
            plsc.addupdate_scatter(acc_v, [bi], dx * f)
            plsc.addupdate_scatter(acc_v, [bi + 1], dy * f)
            plsc.addupdate_scatter(acc_v, [bi + 2], dz * f)

        pltpu.sync_copy(acc_v, out_hbm.at[wid])

    return sc_edge


def _reduce_body(p0_ref, p1_ref, c_ref, o_ref):
    o_ref[...] = (c_ref[...] + jnp.sum(p0_ref[...], axis=0)
                  + jnp.sum(p1_ref[...], axis=0))


def _reduce_partials(partials0, partials1, coords_flat):
    nw, cw = partials0.shape
    return pl.pallas_call(
        _reduce_body,
        in_specs=[
            pl.BlockSpec((nw, cw), lambda: (0, 0)),
            pl.BlockSpec((nw, cw), lambda: (0, 0)),
            pl.BlockSpec((cw,), lambda: (0,)),
        ],
        out_specs=pl.BlockSpec((cw,), lambda: (0,)),
        out_shape=jax.ShapeDtypeStruct((cw,), jnp.float32),
    )(partials0, partials1, coords_flat)


def kernel(a_ij, coords, edge_index, W1, b1, W2, b2, Wh):
    e = a_ij.shape[0]
    n = coords.shape[0]
    block_e = 4096
    nb = pl.cdiv(e, block_e)
    nb0 = 66
    e0 = nb0 * block_e
    e1 = e - e0
    coords_flat = coords.reshape(-1)

    att0, i0, j0 = _compute_att(a_ij, edge_index, W1, b1, W2, b2, Wh,
                                block_e, 0, e0)
    att1, i1, j1 = _compute_att(a_ij, edge_index, W1, b1, W2, b2, Wh,
                                block_e, nb0, e1)
    partials0 = _make_sc_edge(n, e0, 32)(coords_flat, i0, j0, att0)
    partials1 = _make_sc_edge(n, e1, 32)(coords_flat, i1, j1, att1)
    out_flat = _reduce_partials(partials0, partials1, coords_flat)
    return out_flat.reshape(n, 3)


# speedup vs baseline: 1.1251x; 1.1251x over previous
import functools

import jax
import jax.numpy as jnp
from jax import lax
from jax.experimental import pallas as pl
from jax.experimental.pallas import tpu as pltpu
from jax.experimental.pallas import tpu_sc as plsc


def _att_body(a_ref, e_ref, w1_ref, b1_ref, w2_ref, b2_ref, wh_ref,
              o_ref, i_ref, j_ref):
    h = jnp.dot(a_ref[...], w1_ref[...], preferred_element_type=jnp.float32)
    h = h + b1_ref[...]
    h = jnp.where(h >= 0.0, h, 0.01 * h)
    v = jnp.dot(w2_ref[...], wh_ref[...], preferred_element_type=jnp.float32)
    c = jnp.sum(b2_ref[...] * wh_ref[...][:, 0])
    ht = h.T
    att = jnp.dot(v.T, ht, preferred_element_type=jnp.float32) + c
    o_ref[...] = att.reshape(att.shape[1])
    i_ref[...] = e_ref[0, :]
    j_ref[...] = e_ref[1, :]


def _compute_att(a_ij, edge_index, W1, b1, W2, b2, Wh, block_e, first_block,
                 chunk_e):
    nb = pl.cdiv(chunk_e, block_e)
    return pl.pallas_call(
        _att_body,
        grid=(nb,),
        in_specs=[
            pl.BlockSpec((block_e, a_ij.shape[1]),
                         lambda g: (g + first_block, 0)),
            pl.BlockSpec((2, block_e), lambda g: (0, g + first_block)),
            pl.BlockSpec(W1.shape, lambda g: (0, 0)),
            pl.BlockSpec(b1.shape, lambda g: (0,)),
            pl.BlockSpec(W2.shape, lambda g: (0, 0)),
            pl.BlockSpec(b2.shape, lambda g: (0,)),
            pl.BlockSpec(Wh.shape, lambda g: (0, 0)),
        ],
        out_specs=[
            pl.BlockSpec((block_e,), lambda g: (g,)),
            pl.BlockSpec((block_e,), lambda g: (g,)),
            pl.BlockSpec((block_e,), lambda g: (g,)),
        ],
        out_shape=[
            jax.ShapeDtypeStruct((chunk_e,), jnp.float32),
            jax.ShapeDtypeStruct((chunk_e,), jnp.int32),
            jax.ShapeDtypeStruct((chunk_e,), jnp.int32),
        ],
    )(a_ij, edge_index, W1, b1, W2, b2, Wh)


_LANES = 16
_MAGIC = 0x5F3759DF


def _rsqrt16(x):
    i = plsc.bitcast(x, jnp.int32)
    i = _MAGIC - lax.shift_right_logical(i, 1)
    y = plsc.bitcast(i, jnp.float32)
    hx = 0.5 * x
    y = y * (1.5 - hx * y * y)
    y = y * (1.5 - hx * y * y)
    return y


def _make_sc_edge(n, e, n_workers):
    ew = e // n_workers
    cw = 3 * n
    mesh = plsc.VectorSubcoreMesh(core_axis_name="c", subcore_axis_name="s")

    @functools.partial(
        pl.kernel,
        mesh=mesh,
        compiler_params=pltpu.CompilerParams(needs_layout_passes=False),
        out_type=jax.ShapeDtypeStruct((n_workers, cw), jnp.float32),
        scratch_types=[
            pltpu.VMEM((cw,), jnp.float32),
            pltpu.VMEM((cw,), jnp.float32),
            pltpu.VMEM((ew,), jnp.int32),
            pltpu.VMEM((ew,), jnp.int32),
            pltpu.VMEM((ew,), jnp.float32),
            pltpu.SemaphoreType.DMA,
            pltpu.SemaphoreType.DMA,
            pltpu.SemaphoreType.DMA,
            pltpu.SemaphoreType.DMA,
        ],
    )
    def sc_edge(coords_hbm, i_hbm, j_hbm, att_hbm, out_hbm,
                coords_v, acc_v, i_v, j_v, att_v, s0, s1, s2, s3):
        cid = lax.axis_index("c")
        sid = lax.axis_index("s")
        wid = sid * 2 + cid
        base = pl.multiple_of(wid * ew, 8)

        c0 = pltpu.async_copy(coords_hbm, coords_v, s0)
        c1 = pltpu.async_copy(i_hbm.at[pl.ds(base, ew)], i_v, s1)
        c2 = pltpu.async_copy(j_hbm.at[pl.ds(base, ew)], j_v, s2)
        c3 = pltpu.async_copy(att_hbm.at[pl.ds(base, ew)], att_v, s3)

        zeros = jnp.zeros((_LANES,), jnp.float32)

        @plsc.parallel_loop(0, cw, _LANES, unroll=8)
        def _(off):
            acc_v[pl.ds(off, _LANES)] = zeros

        c0.wait()
        c1.wait()
        c2.wait()
        c3.wait()

        @plsc.parallel_loop(0, ew, _LANES, unroll=8)
        def _(off):
            iv = i_v[pl.ds(off, _LANES)]
            jv = j_v[pl.ds(off, _LANES)]
            av = att_v[pl.ds(off, _LANES)]
            bi = iv * 3
            bj = jv * 3
            xi = plsc.load_gather(coords_v, [bi])
            yi = plsc.load_gather(coords_v, [bi + 1])
            zi = plsc.load_gather(coords_v, [bi + 2])
            xj = plsc.load_gather(coords_v, [bj])
            yj = plsc.load_gather(coords_v, [bj + 1])
            zj = plsc.load_gather(coords_v, [bj + 2])
            dx = xi - xj
            dy = yi - yj
            dz = zi - zj
            s2 = dx * dx + dy * dy + dz * dz
            s2 = jnp.maximum(s2, 1e-24)
            f = av * _rsqrt16(s2)
            plsc.addupdate_scatter(acc_v, [bi], dx * f)
            plsc.addupdate_scatter(acc_v, [bi + 1], dy * f)
            plsc.addupdate_scatter(acc_v, [bi + 2], dz * f)

        pltpu.sync_copy(acc_v, out_hbm.at[wid])

    return sc_edge


def _reduce_body(p0_ref, p1_ref, c_ref, o_ref):
    o_ref[...] = (c_ref[...] + jnp.sum(p0_ref[...], axis=0)
                  + jnp.sum(p1_ref[...], axis=0))


def _reduce_partials(partials0, partials1, coords_flat):
    nw, cw = partials0.shape
    return pl.pallas_call(
        _reduce_body,
        in_specs=[
            pl.BlockSpec((nw, cw), lambda: (0, 0)),
            pl.BlockSpec((nw, cw), lambda: (0, 0)),
            pl.BlockSpec((cw,), lambda: (0,)),
        ],
        out_specs=pl.BlockSpec((cw,), lambda: (0,)),
        out_shape=jax.ShapeDtypeStruct((cw,), jnp.float32),
    )(partials0, partials1, coords_flat)


def kernel(a_ij, coords, edge_index, W1, b1, W2, b2, Wh):
    e = a_ij.shape[0]
    n = coords.shape[0]
    block_e = 8192
    nb = pl.cdiv(e, block_e)
    nb0 = 34
    e0 = nb0 * block_e
    e1 = e - e0
    coords_flat = coords.reshape(-1)

    att0, i0, j0 = _compute_att(a_ij, edge_index, W1, b1, W2, b2, Wh,
                                block_e, 0, e0)
    att1, i1, j1 = _compute_att(a_ij, edge_index, W1, b1, W2, b2, Wh,
                                block_e, nb0, e1)
    partials0 = _make_sc_edge(n, e0, 32)(coords_flat, i0, j0, att0)
    partials1 = _make_sc_edge(n, e1, 32)(coords_flat, i1, j1, att1)
    out_flat = _reduce_partials(partials0, partials1, coords_flat)
    return out_flat.reshape(n, 3)


# speedup vs baseline: 1.2026x; 1.0689x over previous
import functools

import jax
import jax.numpy as jnp
from jax import lax
from jax.experimental import pallas as pl
from jax.experimental.pallas import tpu as pltpu
from jax.experimental.pallas import tpu_sc as plsc


def _att_body(a_ref, e_ref, w1_ref, b1_ref, w2_ref, b2_ref, wh_ref,
              o_ref, i_ref, j_ref):
    h = jnp.dot(a_ref[...], w1_ref[...], preferred_element_type=jnp.float32)
    h = h + b1_ref[...]
    h = jnp.where(h >= 0.0, h, 0.01 * h)
    v = jnp.dot(w2_ref[...], wh_ref[...], preferred_element_type=jnp.float32)
    c = jnp.sum(b2_ref[...] * wh_ref[...][:, 0])
    ht = h.T
    att = jnp.dot(v.T, ht, preferred_element_type=jnp.float32) + c
    o_ref[...] = att.reshape(att.shape[1])
    i_ref[...] = e_ref[0, :]
    j_ref[...] = e_ref[1, :]


def _compute_att(a_ij, edge_index, W1, b1, W2, b2, Wh, block_e, first_block,
                 chunk_e):
    nb = pl.cdiv(chunk_e, block_e)
    return pl.pallas_call(
        _att_body,
        grid=(nb,),
        in_specs=[
            pl.BlockSpec((block_e, a_ij.shape[1]),
                         lambda g: (g + first_block, 0)),
            pl.BlockSpec((2, block_e), lambda g: (0, g + first_block)),
            pl.BlockSpec(W1.shape, lambda g: (0, 0)),
            pl.BlockSpec(b1.shape, lambda g: (0,)),
            pl.BlockSpec(W2.shape, lambda g: (0, 0)),
            pl.BlockSpec(b2.shape, lambda g: (0,)),
            pl.BlockSpec(Wh.shape, lambda g: (0, 0)),
        ],
        out_specs=[
            pl.BlockSpec((block_e,), lambda g: (g,)),
            pl.BlockSpec((block_e,), lambda g: (g,)),
            pl.BlockSpec((block_e,), lambda g: (g,)),
        ],
        out_shape=[
            jax.ShapeDtypeStruct((chunk_e,), jnp.float32),
            jax.ShapeDtypeStruct((chunk_e,), jnp.int32),
            jax.ShapeDtypeStruct((chunk_e,), jnp.int32),
        ],
    )(a_ij, edge_index, W1, b1, W2, b2, Wh)


_LANES = 16
_MAGIC = 0x5F3759DF


def _rsqrt16(x):
    i = plsc.bitcast(x, jnp.int32)
    i = _MAGIC - lax.shift_right_logical(i, 1)
    y = plsc.bitcast(i, jnp.float32)
    hx = 0.5 * x
    y = y * (1.5 - hx * y * y)
    y = y * (1.5 - hx * y * y)
    return y


def _make_sc_edge(n, e, n_workers):
    ew = e // n_workers
    cw = 3 * n
    mesh = plsc.VectorSubcoreMesh(core_axis_name="c", subcore_axis_name="s")

    @functools.partial(
        pl.kernel,
        mesh=mesh,
        compiler_params=pltpu.CompilerParams(needs_layout_passes=False),
        out_type=jax.ShapeDtypeStruct((n_workers, cw), jnp.float32),
        scratch_types=[
            pltpu.VMEM((cw,), jnp.float32),
            pltpu.VMEM((cw,), jnp.float32),
            pltpu.VMEM((ew,), jnp.int32),
            pltpu.VMEM((ew,), jnp.int32),
            pltpu.VMEM((ew,), jnp.float32),
            pltpu.SemaphoreType.DMA,
            pltpu.SemaphoreType.DMA,
            pltpu.SemaphoreType.DMA,
            pltpu.SemaphoreType.DMA,
        ],
    )
    def sc_edge(coords_hbm, i_hbm, j_hbm, att_hbm, out_hbm,
                coords_v, acc_v, i_v, j_v, att_v, s0, s1, s2, s3):
        cid = lax.axis_index("c")
        sid = lax.axis_index("s")
        wid = sid * 2 + cid
        base = pl.multiple_of(wid * ew, 8)

        c0 = pltpu.async_copy(coords_hbm, coords_v, s0)
        c1 = pltpu.async_copy(i_hbm.at[pl.ds(base, ew)], i_v, s1)
        c2 = pltpu.async_copy(j_hbm.at[pl.ds(base, ew)], j_v, s2)
        c3 = pltpu.async_copy(att_hbm.at[pl.ds(base, ew)], att_v, s3)

        zeros = jnp.zeros((_LANES,), jnp.float32)

        @plsc.parallel_loop(0, cw, _LANES, unroll=8)
        def _(off):
            acc_v[pl.ds(off, _LANES)] = zeros

        c0.wait()
        c1.wait()
        c2.wait()
        c3.wait()

        @plsc.parallel_loop(0, ew, _LANES, unroll=8)
        def _(off):
            iv = i_v[pl.ds(off, _LANES)]
            jv = j_v[pl.ds(off, _LANES)]
            av = att_v[pl.ds(off, _LANES)]
            bi = iv * 3
            bj = jv * 3
            xi = plsc.load_gather(coords_v, [bi])
            yi = plsc.load_gather(coords_v, [bi + 1])
            zi = plsc.load_gather(coords_v, [bi + 2])
            xj = plsc.load_gather(coords_v, [bj])
            yj = plsc.load_gather(coords_v, [bj + 1])
            zj = plsc.load_gather(coords_v, [bj + 2])
            dx = xi - xj
            dy = yi - yj
            dz = zi - zj
            s2 = dx * dx + dy * dy + dz * dz
            s2 = jnp.maximum(s2, 1e-24)
            f = av * _rsqrt16(s2)
            plsc.addupdate_scatter(acc_v, [bi], dx * f)
            plsc.addupdate_scatter(acc_v, [bi + 1], dy * f)
            plsc.addupdate_scatter(acc_v, [bi + 2], dz * f)

        pltpu.sync_copy(acc_v, out_hbm.at[wid])

    return sc_edge


def _reduce_body(p0_ref, p1_ref, c_ref, o_ref):
    o_ref[...] = (c_ref[...] + jnp.sum(p0_ref[...], axis=0)
                  + jnp.sum(p1_ref[...], axis=0))


def _reduce_partials(partials0, partials1, coords_flat):
    nw, cw = partials0.shape
    return pl.pallas_call(
        _reduce_body,
        in_specs=[
            pl.BlockSpec((nw, cw), lambda: (0, 0)),
            pl.BlockSpec((nw, cw), lambda: (0, 0)),
            pl.BlockSpec((cw,), lambda: (0,)),
        ],
        out_specs=pl.BlockSpec((cw,), lambda: (0,)),
        out_shape=jax.ShapeDtypeStruct((cw,), jnp.float32),
    )(partials0, partials1, coords_flat)


def kernel(a_ij, coords, edge_index, W1, b1, W2, b2, Wh):
    e = a_ij.shape[0]
    n = coords.shape[0]
    block_e = 16384
    nb = pl.cdiv(e, block_e)
    nb0 = 17
    e0 = nb0 * block_e
    e1 = e - e0
    coords_flat = coords.reshape(-1)

    att0, i0, j0 = _compute_att(a_ij, edge_index, W1, b1, W2, b2, Wh,
                                block_e, 0, e0)
    att1, i1, j1 = _compute_att(a_ij, edge_index, W1, b1, W2, b2, Wh,
                                block_e, nb0, e1)
    partials0 = _make_sc_edge(n, e0, 32)(coords_flat, i0, j0, att0)
    partials1 = _make_sc_edge(n, e1, 32)(coords_flat, i1, j1, att1)
    out_flat = _reduce_partials(partials0, partials1, coords_flat)
    return out_flat.reshape(n, 3)


# speedup vs baseline: 1.2282x; 1.0213x over previous
import functools

import jax
import jax.numpy as jnp
from jax import lax
from jax.experimental import pallas as pl
from jax.experimental.pallas import tpu as pltpu
from jax.experimental.pallas import tpu_sc as plsc


def _att_body(a_ref, e_ref, w1_ref, b1_ref, w2_ref, b2_ref, wh_ref,
              o_ref, i_ref, j_ref):
    h = jnp.dot(a_ref[...], w1_ref[...], preferred_element_type=jnp.float32)
    h = h + b1_ref[...]
    h = jnp.where(h >= 0.0, h, 0.01 * h)
    v = jnp.dot(w2_ref[...], wh_ref[...], preferred_element_type=jnp.float32)
    c = jnp.sum(b2_ref[...] * wh_ref[...][:, 0])
    ht = h.T
    att = jnp.dot(v.T, ht, preferred_element_type=jnp.float32) + c
    o_ref[...] = att.reshape(att.shape[1])
    i_ref[...] = e_ref[0, :]
    j_ref[...] = e_ref[1, :]


def _compute_att(a_ij, edge_index, W1, b1, W2, b2, Wh, block_e, first_block,
                 chunk_e):
    nb = pl.cdiv(chunk_e, block_e)
    return pl.pallas_call(
        _att_body,
        grid=(nb,),
        in_specs=[
            pl.BlockSpec((block_e, a_ij.shape[1]),
                         lambda g: (g + first_block, 0)),
            pl.BlockSpec((2, block_e), lambda g: (0, g + first_block)),
            pl.BlockSpec(W1.shape, lambda g: (0, 0)),
            pl.BlockSpec(b1.shape, lambda g: (0,)),
            pl.BlockSpec(W2.shape, lambda g: (0, 0)),
            pl.BlockSpec(b2.shape, lambda g: (0,)),
            pl.BlockSpec(Wh.shape, lambda g: (0, 0)),
        ],
        out_specs=[
            pl.BlockSpec((block_e,), lambda g: (g,)),
            pl.BlockSpec((block_e,), lambda g: (g,)),
            pl.BlockSpec((block_e,), lambda g: (g,)),
        ],
        out_shape=[
            jax.ShapeDtypeStruct((chunk_e,), jnp.float32),
            jax.ShapeDtypeStruct((chunk_e,), jnp.int32),
            jax.ShapeDtypeStruct((chunk_e,), jnp.int32),
        ],
    )(a_ij, edge_index, W1, b1, W2, b2, Wh)


_LANES = 16
_MAGIC = 0x5F3759DF


def _rsqrt16(x):
    i = plsc.bitcast(x, jnp.int32)
    i = _MAGIC - lax.shift_right_logical(i, 1)
    y = plsc.bitcast(i, jnp.float32)
    hx = 0.5 * x
    y = y * (1.5 - hx * y * y)
    y = y * (1.5 - hx * y * y)
    return y


def _make_sc_edge(n, e, n_workers):
    ew = e // n_workers
    cw = 3 * n
    mesh = plsc.VectorSubcoreMesh(core_axis_name="c", subcore_axis_name="s")

    @functools.partial(
        pl.kernel,
        mesh=mesh,
        compiler_params=pltpu.CompilerParams(needs_layout_passes=False),
        out_type=jax.ShapeDtypeStruct((n_workers, cw), jnp.float32),
        scratch_types=[
            pltpu.VMEM((cw,), jnp.float32),
            pltpu.VMEM((cw,), jnp.float32),
            pltpu.VMEM((ew,), jnp.int32),
            pltpu.VMEM((ew,), jnp.int32),
            pltpu.VMEM((ew,), jnp.float32),
            pltpu.SemaphoreType.DMA,
            pltpu.SemaphoreType.DMA,
            pltpu.SemaphoreType.DMA,
            pltpu.SemaphoreType.DMA,
        ],
    )
    def sc_edge(coords_hbm, i_hbm, j_hbm, att_hbm, out_hbm,
                coords_v, acc_v, i_v, j_v, att_v, s0, s1, s2, s3):
        cid = lax.axis_index("c")
        sid = lax.axis_index("s")
        wid = sid * 2 + cid
        base = pl.multiple_of(wid * ew, 8)

        c0 = pltpu.async_copy(coords_hbm, coords_v, s0)
        c1 = pltpu.async_copy(i_hbm.at[pl.ds(base, ew)], i_v, s1)
        c2 = pltpu.async_copy(j_hbm.at[pl.ds(base, ew)], j_v, s2)
        c3 = pltpu.async_copy(att_hbm.at[pl.ds(base, ew)], att_v, s3)

        zeros = jnp.zeros((_LANES,), jnp.float32)

        @plsc.parallel_loop(0, cw, _LANES, unroll=8)
        def _(off):
            acc_v[pl.ds(off, _LANES)] = zeros

        c0.wait()
        c1.wait()
        c2.wait()
        c3.wait()

        @plsc.parallel_loop(0, ew, _LANES, unroll=8)
        def _(off):
            iv = i_v[pl.ds(off, _LANES)]
            jv = j_v[pl.ds(off, _LANES)]
            av = att_v[pl.ds(off, _LANES)]
            bi = iv * 3
            bj = jv * 3
            xi = plsc.load_gather(coords_v, [bi])
            yi = plsc.load_gather(coords_v, [bi + 1])
            zi = plsc.load_gather(coords_v, [bi + 2])
            xj = plsc.load_gather(coords_v, [bj])
            yj = plsc.load_gather(coords_v, [bj + 1])
            zj = plsc.load_gather(coords_v, [bj + 2])
            dx = xi - xj
            dy = yi - yj
            dz = zi - zj
            s2 = dx * dx + dy * dy + dz * dz
            s2 = jnp.maximum(s2, 1e-24)
            f = av * _rsqrt16(s2)
            plsc.addupdate_scatter(acc_v, [bi], dx * f)
            plsc.addupdate_scatter(acc_v, [bi + 1], dy * f)
            plsc.addupdate_scatter(acc_v, [bi + 2], dz * f)

        pltpu.sync_copy(acc_v, out_hbm.at[wid])

    return sc_edge


def _reduce_body(p0_ref, p1_ref, c_ref, o_ref):
    o_ref[...] = (c_ref[...] + jnp.sum(p0_ref[...], axis=0)
                  + jnp.sum(p1_ref[...], axis=0))


def _reduce_partials(partials0, partials1, coords_flat):
    nw, cw = partials0.shape
    return pl.pallas_call(
        _reduce_body,
        in_specs=[
            pl.BlockSpec((nw, cw), lambda: (0, 0)),
            pl.BlockSpec((nw, cw), lambda: (0, 0)),
            pl.BlockSpec((cw,), lambda: (0,)),
        ],
        out_specs=pl.BlockSpec((cw,), lambda: (0,)),
        out_shape=jax.ShapeDtypeStruct((cw,), jnp.float32),
    )(partials0, partials1, coords_flat)


def kernel(a_ij, coords, edge_index, W1, b1, W2, b2, Wh):
    e = a_ij.shape[0]
    n = coords.shape[0]
    block_e = 32768
    nb = pl.cdiv(e, block_e)
    nb0 = 8
    e0 = nb0 * block_e
    e1 = e - e0
    coords_flat = coords.reshape(-1)

    att0, i0, j0 = _compute_att(a_ij, edge_index, W1, b1, W2, b2, Wh,
                                block_e, 0, e0)
    att1, i1, j1 = _compute_att(a_ij, edge_index, W1, b1, W2, b2, Wh,
                                block_e, nb0, e1)
    partials0 = _make_sc_edge(n, e0, 32)(coords_flat, i0, j0, att0)
    partials1 = _make_sc_edge(n, e1, 32)(coords_flat, i1, j1, att1)
    out_flat = _reduce_partials(partials0, partials1, coords_flat)
    return out_flat.reshape(n, 3)


# speedup vs baseline: 1.2368x; 1.0070x over previous
import functools

import jax
import jax.numpy as jnp
from jax import lax
from jax.experimental import pallas as pl
from jax.experimental.pallas import tpu as pltpu
from jax.experimental.pallas import tpu_sc as plsc


def _att_body(a_ref, e_ref, w1_ref, b1_ref, w2_ref, b2_ref, wh_ref,
              o_ref, i_ref, j_ref):
    h = jnp.dot(a_ref[...], w1_ref[...], preferred_element_type=jnp.float32)
    h = h + b1_ref[...]
    h = jnp.where(h >= 0.0, h, 0.01 * h)
    v = jnp.dot(w2_ref[...], wh_ref[...], preferred_element_type=jnp.float32)
    c = jnp.sum(b2_ref[...] * wh_ref[...][:, 0])
    ht = h.T
    att = jnp.dot(v.T, ht, preferred_element_type=jnp.float32) + c
    o_ref[...] = att.reshape(att.shape[1])
    i_ref[...] = e_ref[0, :]
    j_ref[...] = e_ref[1, :]


def _compute_att(a_ij, edge_index, W1, b1, W2, b2, Wh, block_e, first_block,
                 chunk_e):
    nb = pl.cdiv(chunk_e, block_e)
    return pl.pallas_call(
        _att_body,
        grid=(nb,),
        in_specs=[
            pl.BlockSpec((block_e, a_ij.shape[1]),
                         lambda g: (g + first_block, 0)),
            pl.BlockSpec((2, block_e), lambda g: (0, g + first_block)),
            pl.BlockSpec(W1.shape, lambda g: (0, 0)),
            pl.BlockSpec(b1.shape, lambda g: (0,)),
            pl.BlockSpec(W2.shape, lambda g: (0, 0)),
            pl.BlockSpec(b2.shape, lambda g: (0,)),
            pl.BlockSpec(Wh.shape, lambda g: (0, 0)),
        ],
        out_specs=[
            pl.BlockSpec((block_e,), lambda g: (g,)),
            pl.BlockSpec((block_e,), lambda g: (g,)),
            pl.BlockSpec((block_e,), lambda g: (g,)),
        ],
        out_shape=[
            jax.ShapeDtypeStruct((chunk_e,), jnp.float32),
            jax.ShapeDtypeStruct((chunk_e,), jnp.int32),
            jax.ShapeDtypeStruct((chunk_e,), jnp.int32),
        ],
    )(a_ij, edge_index, W1, b1, W2, b2, Wh)


_LANES = 16
_MAGIC = 0x5F3759DF


def _rsqrt16(x):
    i = plsc.bitcast(x, jnp.int32)
    i = _MAGIC - lax.shift_right_logical(i, 1)
    y = plsc.bitcast(i, jnp.float32)
    hx = 0.5 * x
    y = y * (1.5 - hx * y * y)
    y = y * (1.5 - hx * y * y)
    return y


def _make_sc_edge(n, e, n_workers):
    ew = e // n_workers
    cw = 3 * n
    mesh = plsc.VectorSubcoreMesh(core_axis_name="c", subcore_axis_name="s")

    @functools.partial(
        pl.kernel,
        mesh=mesh,
        compiler_params=pltpu.CompilerParams(needs_layout_passes=False),
        out_type=jax.ShapeDtypeStruct((n_workers, cw), jnp.float32),
        scratch_types=[
            pltpu.VMEM((cw,), jnp.float32),
            pltpu.VMEM((cw,), jnp.float32),
            pltpu.VMEM((ew,), jnp.int32),
            pltpu.VMEM((ew,), jnp.int32),
            pltpu.VMEM((ew,), jnp.float32),
            pltpu.SemaphoreType.DMA,
            pltpu.SemaphoreType.DMA,
            pltpu.SemaphoreType.DMA,
            pltpu.SemaphoreType.DMA,
        ],
    )
    def sc_edge(coords_hbm, i_hbm, j_hbm, att_hbm, out_hbm,
                coords_v, acc_v, i_v, j_v, att_v, s0, s1, s2, s3):
        cid = lax.axis_index("c")
        sid = lax.axis_index("s")
        wid = sid * 2 + cid
        base = pl.multiple_of(wid * ew, 8)

        c0 = pltpu.async_copy(coords_hbm, coords_v, s0)
        c1 = pltpu.async_copy(i_hbm.at[pl.ds(base, ew)], i_v, s1)
        c2 = pltpu.async_copy(j_hbm.at[pl.ds(base, ew)], j_v, s2)
        c3 = pltpu.async_copy(att_hbm.at[pl.ds(base, ew)], att_v, s3)

        zeros = jnp.zeros((_LANES,), jnp.float32)

        @plsc.parallel_loop(0, cw, _LANES, unroll=8)
        def _(off):
            acc_v[pl.ds(off, _LANES)] = zeros

        c0.wait()
        c1.wait()
        c2.wait()
        c3.wait()

        @plsc.parallel_loop(0, ew, _LANES, unroll=8)
        def _(off):
            iv = i_v[pl.ds(off, _LANES)]
            jv = j_v[pl.ds(off, _LANES)]
            av = att_v[pl.ds(off, _LANES)]
            bi = iv * 3
            bj = jv * 3
            xi = plsc.load_gather(coords_v, [bi])
            yi = plsc.load_gather(coords_v, [bi + 1])
            zi = plsc.load_gather(coords_v, [bi + 2])
            xj = plsc.load_gather(coords_v, [bj])
            yj = plsc.load_gather(coords_v, [bj + 1])
            zj = plsc.load_gather(coords_v, [bj + 2])
            dx = xi - xj
            dy = yi - yj
            dz = zi - zj
            s2 = dx * dx + dy * dy + dz * dz
            s2 = jnp.maximum(s2, 1e-24)
            f = av * _rsqrt16(s2)
            plsc.addupdate_scatter(acc_v, [bi], dx * f)
            plsc.addupdate_scatter(acc_v, [bi + 1], dy * f)
            plsc.addupdate_scatter(acc_v, [bi + 2], dz * f)

        pltpu.sync_copy(acc_v, out_hbm.at[wid])

    return sc_edge


def _reduce_body(p0_ref, p1_ref, c_ref, o_ref):
    o_ref[...] = (c_ref[...] + jnp.sum(p0_ref[...], axis=0)
                  + jnp.sum(p1_ref[...], axis=0))


def _reduce_partials(partials0, partials1, coords_flat):
    nw, cw = partials0.shape
    bw = 8192
    return pl.pallas_call(
        _reduce_body,
        grid=(pl.cdiv(cw, bw),),
        in_specs=[
            pl.BlockSpec((nw, bw), lambda g: (0, g)),
            pl.BlockSpec((nw, bw), lambda g: (0, g)),
            pl.BlockSpec((bw,), lambda g: (g,)),
        ],
        out_specs=pl.BlockSpec((bw,), lambda g: (g,)),
        out_shape=jax.ShapeDtypeStruct((cw,), jnp.float32),
    )(partials0, partials1, coords_flat)


def kernel(a_ij, coords, edge_index, W1, b1, W2, b2, Wh):
    e = a_ij.shape[0]
    n = coords.shape[0]
    block_e = 32768
    nb = pl.cdiv(e, block_e)
    nb0 = 8
    e0 = nb0 * block_e
    e1 = e - e0
    coords_flat = coords.reshape(-1)

    att0, i0, j0 = _compute_att(a_ij, edge_index, W1, b1, W2, b2, Wh,
                                block_e, 0, e0)
    att1, i1, j1 = _compute_att(a_ij, edge_index, W1, b1, W2, b2, Wh,
                                block_e, nb0, e1)
    partials0 = _make_sc_edge(n, e0, 32)(coords_flat, i0, j0, att0)
    partials1 = _make_sc_edge(n, e1, 32)(coords_flat, i1, j1, att1)
    out_flat = _reduce_partials(partials0, partials1, coords_flat)
    return out_flat.reshape(n, 3)


# speedup vs baseline: 1.2506x; 1.0112x over previous
import functools

import jax
import jax.numpy as jnp
from jax import lax
from jax.experimental import pallas as pl
from jax.experimental.pallas import tpu as pltpu
from jax.experimental.pallas import tpu_sc as plsc


def _att_body(a_ref, e_ref, w1_ref, b1_ref, w2_ref, b2_ref, wh_ref,
              o_ref, i_ref, j_ref):
    h = jnp.dot(a_ref[...], w1_ref[...], preferred_element_type=jnp.float32)
    h = h + b1_ref[...]
    h = jnp.where(h >= 0.0, h, 0.01 * h)
    v = jnp.dot(w2_ref[...], wh_ref[...], preferred_element_type=jnp.float32)
    c = jnp.sum(b2_ref[...] * wh_ref[...][:, 0])
    ht = h.T
    att = jnp.dot(v.T, ht, preferred_element_type=jnp.float32) + c
    o_ref[...] = att.reshape(att.shape[1])
    i_ref[...] = e_ref[0, :]
    j_ref[...] = e_ref[1, :]


def _compute_att(a_ij, edge_index, W1, b1, W2, b2, Wh, block_e, first_block,
                 chunk_e):
    nb = pl.cdiv(chunk_e, block_e)
    return pl.pallas_call(
        _att_body,
        grid=(nb,),
        in_specs=[
            pl.BlockSpec((block_e, a_ij.shape[1]),
                         lambda g: (g + first_block, 0)),
            pl.BlockSpec((2, block_e), lambda g: (0, g + first_block)),
            pl.BlockSpec(W1.shape, lambda g: (0, 0)),
            pl.BlockSpec(b1.shape, lambda g: (0,)),
            pl.BlockSpec(W2.shape, lambda g: (0, 0)),
            pl.BlockSpec(b2.shape, lambda g: (0,)),
            pl.BlockSpec(Wh.shape, lambda g: (0, 0)),
        ],
        out_specs=[
            pl.BlockSpec((block_e,), lambda g: (g,)),
            pl.BlockSpec((block_e,), lambda g: (g,)),
            pl.BlockSpec((block_e,), lambda g: (g,)),
        ],
        out_shape=[
            jax.ShapeDtypeStruct((chunk_e,), jnp.float32),
            jax.ShapeDtypeStruct((chunk_e,), jnp.int32),
            jax.ShapeDtypeStruct((chunk_e,), jnp.int32),
        ],
    )(a_ij, edge_index, W1, b1, W2, b2, Wh)


_LANES = 16
_MAGIC = 0x5F3759DF


def _rsqrt16(x):
    i = plsc.bitcast(x, jnp.int32)
    i = _MAGIC - lax.shift_right_logical(i, 1)
    y = plsc.bitcast(i, jnp.float32)
    hx = 0.5 * x
    y = y * (1.5 - hx * y * y)
    y = y * (1.5 - hx * y * y)
    return y


def _make_sc_edge(n, e, n_workers):
    ew = e // n_workers
    cw = 3 * n
    mesh = plsc.VectorSubcoreMesh(core_axis_name="c", subcore_axis_name="s")

    @functools.partial(
        pl.kernel,
        mesh=mesh,
        compiler_params=pltpu.CompilerParams(needs_layout_passes=False),
        out_type=jax.ShapeDtypeStruct((n_workers, cw), jnp.float32),
        scratch_types=[
            pltpu.VMEM((cw,), jnp.float32),
            pltpu.VMEM((cw,), jnp.float32),
            pltpu.VMEM((ew,), jnp.int32),
            pltpu.VMEM((ew,), jnp.int32),
            pltpu.VMEM((ew,), jnp.float32),
            pltpu.SemaphoreType.DMA,
            pltpu.SemaphoreType.DMA,
            pltpu.SemaphoreType.DMA,
            pltpu.SemaphoreType.DMA,
        ],
    )
    def sc_edge(coords_hbm, i_hbm, j_hbm, att_hbm, out_hbm,
                coords_v, acc_v, i_v, j_v, att_v, s0, s1, s2, s3):
        cid = lax.axis_index("c")
        sid = lax.axis_index("s")
        wid = sid * 2 + cid
        base = pl.multiple_of(wid * ew, 8)

        c0 = pltpu.async_copy(coords_hbm, coords_v, s0)
        c1 = pltpu.async_copy(i_hbm.at[pl.ds(base, ew)], i_v, s1)
        c2 = pltpu.async_copy(j_hbm.at[pl.ds(base, ew)], j_v, s2)
        c3 = pltpu.async_copy(att_hbm.at[pl.ds(base, ew)], att_v, s3)

        zeros = jnp.zeros((_LANES,), jnp.float32)

        @plsc.parallel_loop(0, cw, _LANES, unroll=8)
        def _(off):
            acc_v[pl.ds(off, _LANES)] = zeros

        c0.wait()
        c1.wait()
        c2.wait()
        c3.wait()

        @plsc.parallel_loop(0, ew, _LANES, unroll=8)
        def _(off):
            iv = i_v[pl.ds(off, _LANES)]
            jv = j_v[pl.ds(off, _LANES)]
            av = att_v[pl.ds(off, _LANES)]
            bi = iv * 3
            bj = jv * 3
            xi = plsc.load_gather(coords_v, [bi])
            yi = plsc.load_gather(coords_v, [bi + 1])
            zi = plsc.load_gather(coords_v, [bi + 2])
            xj = plsc.load_gather(coords_v, [bj])
            yj = plsc.load_gather(coords_v, [bj + 1])
            zj = plsc.load_gather(coords_v, [bj + 2])
            dx = xi - xj
            dy = yi - yj
            dz = zi - zj
            s2 = dx * dx + dy * dy + dz * dz
            s2 = jnp.maximum(s2, 1e-24)
            f = av * _rsqrt16(s2)
            plsc.addupdate_scatter(acc_v, [bi], dx * f)
            plsc.addupdate_scatter(acc_v, [bi + 1], dy * f)
            plsc.addupdate_scatter(acc_v, [bi + 2], dz * f)

        pltpu.sync_copy(acc_v, out_hbm.at[wid])

    return sc_edge


def _reduce_body(p0_ref, p1_ref, c_ref, o_ref):
    o_ref[...] = (c_ref[...] + jnp.sum(p0_ref[...], axis=0)
                  + jnp.sum(p1_ref[...], axis=0))


def _reduce_partials(partials0, partials1, coords_flat):
    nw, cw = partials0.shape
    bw = 8192
    return pl.pallas_call(
        _reduce_body,
        grid=(pl.cdiv(cw, bw),),
        in_specs=[
            pl.BlockSpec((nw, bw), lambda g: (0, g)),
            pl.BlockSpec((nw, bw), lambda g: (0, g)),
            pl.BlockSpec((bw,), lambda g: (g,)),
        ],
        out_specs=pl.BlockSpec((bw,), lambda g: (g,)),
        out_shape=jax.ShapeDtypeStruct((cw,), jnp.float32),
    )(partials0, partials1, coords_flat)


def kernel(a_ij, coords, edge_index, W1, b1, W2, b2, Wh):
    e = a_ij.shape[0]
    n = coords.shape[0]
    be0 = 49152
    be1 = 16384
    e0 = 5 * be0
    e1 = e - e0
    coords_flat = coords.reshape(-1)

    att0, i0, j0 = _compute_att(a_ij, edge_index, W1, b1, W2, b2, Wh,
                                be0, 0, e0)
    att1, i1, j1 = _compute_att(a_ij, edge_index, W1, b1, W2, b2, Wh,
                                be1, e0 // be1, e1)
    partials0 = _make_sc_edge(n, e0, 32)(coords_flat, i0, j0, att0)
    partials1 = _make_sc_edge(n, e1, 32)(coords_flat, i1, j1, att1)
    out_flat = _reduce_partials(partials0, partials1, coords_flat)
    return out_flat.reshape(n, 3)
